# Initial kernel scaffold; baseline (speedup 1.0000x reference)
import jax
import jax.numpy as jnp
from jax.experimental import pallas as pl


def kernel(text, text_len, table):
    raise NotImplementedError("write your pallas kernel here")


# speedup vs baseline: 1.8953x; 1.8953x over previous
import functools

import jax
import jax.numpy as jnp
from jax import lax
from jax.experimental import pallas as pl
from jax.experimental.pallas import tpu as pltpu
from jax.experimental.pallas import tpu_sc as plsc


def _build_kernel(B, L, V, D):
    info = plsc.get_sparse_core_info()
    NC, NS, NL = info.num_cores, info.num_subcores, info.num_lanes
    NW = NC * NS
    assert B % NW == 0
    SPW = B // NW
    C = 16
    assert SPW % C == 0
    NCH = SPW // C
    CL = C * L
    assert D % NL == 0
    DG = D // NL
    pieces = []
    off = 0
    while off < CL:
        n = min(128, CL - off)
        pieces.append((off, n))
        off += n

    mesh = plsc.VectorSubcoreMesh(core_axis_name="c", subcore_axis_name="s")

    @functools.partial(
        pl.kernel,
        mesh=mesh,
        compiler_params=pltpu.CompilerParams(use_tc_tiling_on_sc=False),
        out_type=(
            jax.ShapeDtypeStruct((B * L, D), jnp.float32),
            jax.ShapeDtypeStruct((B, D), jnp.float32),
        ),
        scratch_types=[
            pltpu.VMEM((CL,), jnp.int32),
            pltpu.VMEM((CL, D), jnp.float32),
            pltpu.VMEM((SPW,), jnp.int32),
            pltpu.VMEM((C, D), jnp.float32),
            pltpu.SemaphoreType.DMA,
        ],
    )
    def sc_kernel(text_ref, len_ref, table_ref, embs_ref, semb_ref,
                  idx_v, rows_v, lens_v, semb_v, sem):
        wid = lax.axis_index("s") * NC + lax.axis_index("c")
        wbase = wid * SPW
        pltpu.sync_copy(len_ref.at[pl.ds(wbase, SPW)], lens_v)
        lane = lax.broadcasted_iota(jnp.int32, (NL,), 0)

        def chunk_body(ci, carry):
            s0 = wbase + ci * C
            pltpu.sync_copy(text_ref.at[pl.ds(s0 * L, CL)], idx_v)
            cps = [
                pltpu.async_copy(
                    table_ref.at[idx_v.at[pl.ds(o, n)]],
                    rows_v.at[pl.ds(o, n)],
                    sem,
                )
                for (o, n) in pieces
            ]
            for cp in cps:
                cp.wait()
            pltpu.sync_copy(rows_v, embs_ref.at[pl.ds(s0 * L, CL)])
            lens16 = lens_v[pl.ds(ci * C, C)]
            for j in range(C):
                lenj = lens16[j]
                lenf = lenj.astype(jnp.float32)
                rb = j * L

                def ibody(i, accs):
                    r = rb + i
                    return tuple(
                        accs[g] + rows_v[r, pl.ds(g * NL, NL)]
                        for g in range(DG)
                    )

                z = jnp.zeros((NL,), jnp.float32)
                accs = lax.fori_loop(0, lenj, ibody, (z,) * DG)
                for g in range(DG):
                    semb_v[j, pl.ds(g * NL, NL)] = accs[g] / lenf
            pltpu.sync_copy(semb_v, semb_ref.at[pl.ds(s0, C)])
            return carry

        lax.fori_loop(0, NCH, chunk_body, 0)

    return sc_kernel


def kernel(text, text_len, table):
    B, L = text.shape
    V, D = table.shape
    sc = _build_kernel(B, L, V, D)
    embs_flat, semb = sc(text.reshape(B * L), text_len, table)
    return embs_flat.reshape(B, L, D), semb
